# baseline (speedup 1.0000x reference)
import functools
import math

import jax
import jax.numpy as jnp
from jax import lax
from jax.experimental import pallas as pl
from jax.experimental.pallas import tpu as pltpu
from jax.experimental.pallas import tpu_sc as plsc

_B, _N, _D = 128, 577, 768
_P = _N - 1
_S = 16
_K = _P // 2
_NSEL = _K + 1
_PAD = 320
_CH = 64
_NCH = _PAD // _CH
_LAST = _NSEL - (_NCH - 1) * _CH
_SCALE = math.sqrt(_N / _NSEL)
_VPR = _P // 16
_EPS = 1e-10
_INV_TAU = 2.0


def _scores_body(logits_ref, u_ref, out_ref):
    lg = logits_ref[...]
    acc = jnp.zeros_like(lg)
    for s in range(_S):
        u = u_ref[s]
        g = -jnp.log(-jnp.log(u + _EPS) + _EPS)
        z = (lg + g) * _INV_TAU
        z = z - jnp.max(z, axis=-1, keepdims=True)
        e = jnp.exp(z)
        acc = acc + e / jnp.sum(e, axis=-1, keepdims=True)
    out_ref[...] = acc * (1.0 / _S)


_scores_call = pl.pallas_call(
    _scores_body,
    out_shape=jax.ShapeDtypeStruct((_B, _P), jnp.float32),
)


def _make_sel_gather():
    nc, ns = 2, 16
    nw = nc * ns
    rows = _B // nw
    mesh = plsc.VectorSubcoreMesh(
        core_axis_name="c", subcore_axis_name="s", num_cores=nc)

    @functools.partial(
        pl.kernel,
        mesh=mesh,
        out_type=jax.ShapeDtypeStruct((_B * _NSEL, _D), jnp.float32),
        scratch_types=[
            pltpu.VMEM((_P,), jnp.float32),
            pltpu.VMEM((_PAD,), jnp.int32),
            pltpu.VMEM((_CH, _D), jnp.float32),
            pltpu.VMEM((_CH, _D), jnp.float32),
            pltpu.SemaphoreType.DMA,
            pltpu.SemaphoreType.DMA,
            pltpu.SemaphoreType.DMA,
            pltpu.SemaphoreType.DMA,
        ],
        compiler_params=pltpu.CompilerParams(
            use_tc_tiling_on_sc=False, needs_layout_passes=False),
    )
    def body(scores_hbm, x_hbm, out_hbm, sc_v, idx_v, bufa, bufb,
             sg0, sg1, ss0, ss1):
        wid = lax.axis_index("s") * nc + lax.axis_index("c")
        bufs = (bufa, bufb)
        gsem = (sg0, sg1)
        ssem = (ss0, ss1)
        ones = jnp.ones((16,), jnp.int32)
        zeros = jnp.zeros((16,), jnp.int32)
        lane = lax.iota(jnp.int32, 16)

        def count_ge(t):
            def cbody(j, acc):
                bits = plsc.bitcast(sc_v[pl.ds(j * 16, 16)], jnp.int32)
                return acc + jnp.where(bits >= t, ones, zeros)
            acc = lax.fori_loop(0, _VPR, cbody, zeros)
            return jnp.sum(acc)

        def row_body(r, carry):
            b = wid * rows + r
            pltpu.sync_copy(scores_hbm.at[b], sc_v)

            def bbody(i, lohi):
                lo, hi = lohi
                mid = lo + ((hi - lo) >> 1)
                big = count_ge(mid) >= _K
                return (jnp.where(big, mid, lo), jnp.where(big, hi, mid))
            lo, _hi = lax.fori_loop(
                0, 31, bbody, (jnp.int32(0), jnp.int32(0x7F800000)))
            thr = lo
            need = _K - count_ge(thr + 1)

            base = b * _N
            idx_v[pl.ds(0, 16)] = jnp.where(lane == 0, base, 0)
            idx_v[pl.ds(_K, 16)] = zeros
            idx_v[pl.ds(_K + 16, 16)] = zeros
            off = jnp.int32(1)
            taken = jnp.int32(0)
            for j in range(_VPR):
                bits = plsc.bitcast(sc_v[pl.ds(j * 16, 16)], jnp.int32)
                gt = bits > thr
                eq = bits == thr
                eqc = plsc.cumsum(jnp.where(eq, ones, zeros))
                take_eq = jnp.logical_and(eq, (taken + eqc) <= need)
                m = jnp.logical_or(gt, take_eq)
                tok = lane + (base + 1 + j * 16)
                plsc.store_compressed(idx_v.at[pl.ds(off, 16)], tok, mask=m)
                off = off + jnp.sum(jnp.where(m, ones, zeros))
                taken = taken + jnp.sum(jnp.where(take_eq, ones, zeros))

            def start_gather(c):
                return pltpu.async_copy(
                    x_hbm.at[idx_v.at[pl.ds(c * _CH, _CH)]],
                    bufs[c % 2], gsem[c % 2])

            def scale(c):
                buf = bufs[c % 2]
                def rb(i, _):
                    for k2 in range(_D // 16):
                        sl = pl.ds(k2 * 16, 16)
                        buf[i, sl] = buf[i, sl] * _SCALE
                    return 0
                lax.fori_loop(0, _CH, rb, 0)

            def start_scatter(c):
                ob = b * _NSEL + c * _CH
                if c == _NCH - 1:
                    return pltpu.async_copy(
                        bufs[c % 2].at[pl.ds(0, _LAST)],
                        out_hbm.at[pl.ds(ob, _LAST)], ssem[c % 2])
                return pltpu.async_copy(
                    bufs[c % 2], out_hbm.at[pl.ds(ob, _CH)], ssem[c % 2])

            hg = [None] * _NCH
            hs = [None] * _NCH
            hg[0] = start_gather(0)
            for c in range(_NCH):
                hg[c].wait()
                scale(c)
                if c >= 1:
                    hs[c - 1].wait()
                hs[c] = start_scatter(c)
                if c + 1 < _NCH:
                    hg[c + 1] = start_gather(c + 1)
            hs[_NCH - 1].wait()
            return carry

        lax.fori_loop(0, rows, row_body, 0)

    return body


_sel_gather_cache = []


def kernel(x, cls_attn, u):
    logits = cls_attn[:, 1:]
    scores = _scores_call(logits, u)
    x2 = x.reshape(_B * _N, _D)
    if not _sel_gather_cache:
        _sel_gather_cache.append(_make_sel_gather())
    out = _sel_gather_cache[0](scores, x2)
    return out.reshape(_B, _NSEL, _D)


# speedup vs baseline: 1.9640x; 1.9640x over previous
import functools
import math

import jax
import jax.numpy as jnp
from jax import lax
from jax.experimental import pallas as pl
from jax.experimental.pallas import tpu as pltpu
from jax.experimental.pallas import tpu_sc as plsc

_B, _N, _D = 128, 577, 768
_P = _N - 1
_S = 16
_K = _P // 2
_NSEL = _K + 1
_PAD = 320
_CH = 64
_NCH = _PAD // _CH
_LAST = _NSEL - (_NCH - 1) * _CH
_LASTP = 40
_SCALE = math.sqrt(_N / _NSEL)
_VPR = _P // 16
_EPS = 1e-10
_INV_TAU = 2.0


def _scores_body(logits_ref, u_ref, out_ref):
    lg = logits_ref[...]
    acc = jnp.zeros_like(lg)
    for s in range(_S):
        u = u_ref[s]
        g = -jnp.log(-jnp.log(u + _EPS) + _EPS)
        z = (lg + g) * _INV_TAU
        z = z - jnp.max(z, axis=-1, keepdims=True)
        e = jnp.exp(z)
        acc = acc + e / jnp.sum(e, axis=-1, keepdims=True)
    out_ref[...] = acc * (1.0 / _S)


_scores_call = pl.pallas_call(
    _scores_body,
    out_shape=jax.ShapeDtypeStruct((_B, _P), jnp.float32),
)


def _make_sel_gather():
    nc, ns = 2, 16
    nw = nc * ns
    rows = _B // nw
    mesh = plsc.VectorSubcoreMesh(
        core_axis_name="c", subcore_axis_name="s", num_cores=nc)

    @functools.partial(
        pl.kernel,
        mesh=mesh,
        out_type=jax.ShapeDtypeStruct((_B, _NSEL, _D), jnp.float32),
        scratch_types=[
            pltpu.VMEM((_P,), jnp.float32),
            pltpu.VMEM((_PAD,), jnp.int32),
            pltpu.VMEM((_CH, _D), jnp.float32),
            pltpu.VMEM((_CH, _D), jnp.float32),
            pltpu.VMEM((_LASTP, _D), jnp.float32),
            pltpu.VMEM((_LASTP,), jnp.int32),
            pltpu.SemaphoreType.DMA,
            pltpu.SemaphoreType.DMA,
            pltpu.SemaphoreType.DMA,
            pltpu.SemaphoreType.DMA,
        ],
        compiler_params=pltpu.CompilerParams(needs_layout_passes=False),
    )
    def body(scores_hbm, x_hbm, out_hbm, sc_v, idx_v, bufa, bufb,
             buf_last, idx_last, sg0, sg1, ss0, ss1):
        wid = lax.axis_index("s") * nc + lax.axis_index("c")
        bufs = (bufa, bufb)
        gsem = (sg0, sg1)
        ssem = (ss0, ss1)
        ones = jnp.ones((16,), jnp.int32)
        zeros = jnp.zeros((16,), jnp.int32)
        lane = lax.iota(jnp.int32, 16)
        idx_last[pl.ds(0, 16)] = lane + (_NCH - 1) * _CH
        idx_last[pl.ds(16, 16)] = lane + ((_NCH - 1) * _CH + 16)
        idx_last[pl.ds(_LASTP - 16, 16)] = jnp.minimum(
            lane + (_LASTP - 16 + (_NCH - 1) * _CH), _NSEL - 1)

        def count_ge(t):
            def cbody(j, acc):
                bits = plsc.bitcast(sc_v[pl.ds(j * 16, 16)], jnp.int32)
                return acc + jnp.where(bits >= t, ones, zeros)
            acc = lax.fori_loop(0, _VPR, cbody, zeros)
            return jnp.sum(acc)

        def row_body(r, carry):
            b = wid * rows + r
            pltpu.sync_copy(scores_hbm.at[pl.ds(b * _P, _P)], sc_v)

            def bbody(i, lohi):
                lo, hi = lohi
                mid = lo + ((hi - lo) >> 1)
                big = count_ge(mid) >= _K
                return (jnp.where(big, mid, lo), jnp.where(big, hi, mid))
            lo, _hi = lax.fori_loop(
                0, 31, bbody, (jnp.int32(0), jnp.int32(0x7F800000)))
            thr = lo
            need = _K - count_ge(thr + 1)

            base = 0
            idx_v[pl.ds(0, 16)] = zeros
            idx_v[pl.ds(_K, 16)] = zeros
            idx_v[pl.ds(_K + 16, 16)] = zeros
            off = jnp.int32(1)
            taken = jnp.int32(0)
            for j in range(_VPR):
                bits = plsc.bitcast(sc_v[pl.ds(j * 16, 16)], jnp.int32)
                gt = bits > thr
                eq = bits == thr
                eqc = plsc.cumsum(jnp.where(eq, ones, zeros))
                take_eq = jnp.logical_and(eq, (taken + eqc) <= need)
                m = jnp.logical_or(gt, take_eq)
                tok = lane + (base + 1 + j * 16)
                plsc.store_compressed(idx_v.at[pl.ds(off, 16)], tok, mask=m)
                off = off + jnp.sum(jnp.where(m, ones, zeros))
                taken = taken + jnp.sum(jnp.where(take_eq, ones, zeros))
            t_last = idx_v[pl.ds(_NSEL - 9, 16)][8]
            idx_v[pl.ds(_NSEL - 1, 16)] = jnp.broadcast_to(t_last, (16,))

            def start_gather(c):
                if c == _NCH - 1:
                    return pltpu.async_copy(
                        x_hbm.at[b].at[idx_v.at[pl.ds((_NCH - 1) * _CH,
                                                      _LASTP)]],
                        buf_last, gsem[c % 2])
                return pltpu.async_copy(
                    x_hbm.at[b].at[idx_v.at[pl.ds(c * _CH, _CH)]],
                    bufs[c % 2], gsem[c % 2])

            def scale(c):
                buf = buf_last if c == _NCH - 1 else bufs[c % 2]
                n = _LASTP if c == _NCH - 1 else _CH
                def rb(i, _):
                    for k2 in range(_D // 16):
                        sl = pl.ds(k2 * 16, 16)
                        buf[i, sl] = buf[i, sl] * _SCALE
                    return 0
                lax.fori_loop(0, n, rb, 0)

            def start_scatter(c):
                if c == _NCH - 1:
                    return pltpu.async_copy(
                        buf_last, out_hbm.at[b].at[idx_last], ssem[c % 2])
                return pltpu.async_copy(
                    bufs[c % 2], out_hbm.at[b].at[pl.ds(c * _CH, _CH)],
                    ssem[c % 2])

            hg = [None] * _NCH
            hs = [None] * _NCH
            hg[0] = start_gather(0)
            for c in range(_NCH):
                hg[c].wait()
                scale(c)
                if c >= 1:
                    hs[c - 1].wait()
                hs[c] = start_scatter(c)
                if c + 1 < _NCH:
                    hg[c + 1] = start_gather(c + 1)
            hs[_NCH - 1].wait()
            return carry

        lax.fori_loop(0, rows, row_body, 0)

    return body


_sel_gather_cache = []


def kernel(x, cls_attn, u):
    logits = cls_attn[:, 1:]
    scores = _scores_call(logits, u).reshape(_B * _P)
    if not _sel_gather_cache:
        _sel_gather_cache.append(_make_sel_gather())
    return _sel_gather_cache[0](scores, x)


# speedup vs baseline: 5.0444x; 2.5685x over previous
import functools
import math

import jax
import jax.numpy as jnp
from jax import lax
from jax.experimental import pallas as pl
from jax.experimental.pallas import tpu as pltpu
from jax.experimental.pallas import tpu_sc as plsc

_B, _N, _D = 128, 577, 768
_P = _N - 1
_S = 16
_K = _P // 2
_NSEL = _K + 1
_PAD = 320
_CH = 64
_NCH = _PAD // _CH
_LASTP = 40
_SCALE = math.sqrt(_N / _NSEL)
_VPR = _P // 16
_EPS = 1e-10
_INV_TAU = 2.0


def _scores_body(logits_ref, u_ref, out_ref):
    lg = logits_ref[...]
    acc = jnp.zeros_like(lg)
    for s in range(_S):
        u = u_ref[s]
        g = -jnp.log(-jnp.log(u + _EPS) + _EPS)
        z = (lg + g) * _INV_TAU
        z = z - jnp.max(z, axis=-1, keepdims=True)
        e = jnp.exp(z)
        acc = acc + e / jnp.sum(e, axis=-1, keepdims=True)
    out_ref[...] = acc * (1.0 / _S)


_scores_call = pl.pallas_call(
    _scores_body,
    out_shape=jax.ShapeDtypeStruct((_B, _P), jnp.float32),
)


def _make_sel_gather():
    nc, ns = 2, 16
    nw = nc * ns
    rows = _B // nw
    mesh = plsc.VectorSubcoreMesh(
        core_axis_name="c", subcore_axis_name="s", num_cores=nc)

    @functools.partial(
        pl.kernel,
        mesh=mesh,
        out_type=jax.ShapeDtypeStruct((_NSEL * _B, _D), jnp.float32),
        scratch_types=[
            pltpu.VMEM((_P,), jnp.float32),
            pltpu.VMEM((_P,), jnp.float32),
            pltpu.VMEM((_PAD,), jnp.int32),
            pltpu.VMEM((_PAD,), jnp.int32),
            pltpu.VMEM((_CH, _D), jnp.float32),
            pltpu.VMEM((_CH, _D), jnp.float32),
        ] + [pltpu.VMEM((_CH,), jnp.int32)] * 8 + [
            pltpu.VMEM((_LASTP,), jnp.int32),
            pltpu.VMEM((_LASTP,), jnp.int32),
            pltpu.SemaphoreType.DMA,
            pltpu.SemaphoreType.DMA,
            pltpu.SemaphoreType.DMA,
            pltpu.SemaphoreType.DMA,
            pltpu.SemaphoreType.DMA,
        ],
        compiler_params=pltpu.CompilerParams(needs_layout_passes=False),
    )
    def body(scores_hbm, x_hbm, out_hbm, sc_a, sc_b, idxa, idxb, bufa, bufb,
             io00, io01, io02, io03, io10, io11, io12, io13,
             iol_a, iol_b, sg0, sg1, ss0, ss1, ssc):
        wid = lax.axis_index("s") * nc + lax.axis_index("c")
        bufs = (bufa, bufb)
        idxs = (idxa, idxb)
        scs = (sc_a, sc_b)
        ios = ((io00, io01, io02, io03), (io10, io11, io12, io13))
        iols = (iol_a, iol_b)
        gsem = (sg0, sg1)
        ssem = (ss0, ss1)
        ones = jnp.ones((16,), jnp.int32)
        zeros = jnp.zeros((16,), jnp.int32)
        lane = lax.iota(jnp.int32, 16)
        lane_b = lane * _B

        def fetch_scores(b, p):
            return pltpu.async_copy(
                scores_hbm.at[pl.ds(b * _P, _P)], scs[p], ssc)

        def count_ge(p, t):
            def cbody(j, acc):
                bits = plsc.bitcast(scs[p][pl.ds(j * 16, 16)], jnp.int32)
                return acc + jnp.where(bits >= t, ones, zeros)
            acc = lax.fori_loop(0, _VPR, cbody, zeros)
            return jnp.sum(acc)

        def select(b, p):
            idx_v = idxs[p]
            for c in range(_NCH - 1):
                for k in range(_CH // 16):
                    ios[p][c][pl.ds(k * 16, 16)] = (
                        lane_b + ((c * _CH + k * 16) * _B + b))
            iols[p][pl.ds(0, 16)] = lane_b + ((_NCH - 1) * _CH * _B + b)
            iols[p][pl.ds(16, 16)] = (
                lane_b + (((_NCH - 1) * _CH + 16) * _B + b))
            iols[p][pl.ds(_LASTP - 16, 16)] = (
                jnp.minimum(lane + (_LASTP - 16 + (_NCH - 1) * _CH),
                            _NSEL - 1) * _B + b)

            def bbody(i, lohi):
                lo, hi = lohi
                mid = lo + ((hi - lo) >> 1)
                big = count_ge(p, mid) >= _K
                return (jnp.where(big, mid, lo), jnp.where(big, hi, mid))
            lo, _hi = lax.fori_loop(
                0, 31, bbody, (jnp.int32(0), jnp.int32(0x7F800000)))
            thr = lo
            need = _K - count_ge(p, thr + 1)

            idx_v[pl.ds(0, 16)] = jnp.where(lane == 0, b, 0)
            idx_v[pl.ds(_K, 16)] = zeros
            idx_v[pl.ds(_K + 16, 16)] = zeros
            off = jnp.int32(1)
            taken = jnp.int32(0)
            for j in range(_VPR):
                bits = plsc.bitcast(scs[p][pl.ds(j * 16, 16)], jnp.int32)
                gt = bits > thr
                eq = bits == thr
                eqc = plsc.cumsum(jnp.where(eq, ones, zeros))
                take_eq = jnp.logical_and(eq, (taken + eqc) <= need)
                m = jnp.logical_or(gt, take_eq)
                tok = lane_b + ((1 + j * 16) * _B + b)
                plsc.store_compressed(idx_v.at[pl.ds(off, 16)], tok, mask=m)
                off = off + jnp.sum(jnp.where(m, ones, zeros))
                taken = taken + jnp.sum(jnp.where(take_eq, ones, zeros))
            t_last = idx_v[pl.ds(_NSEL - 9, 16)][8]
            idx_v[pl.ds(_NSEL - 1, 16)] = jnp.broadcast_to(t_last, (16,))

        def transfers(p, mid_work):
            idx_v = idxs[p]

            def start_gather(c):
                if c == _NCH - 1:
                    return pltpu.async_copy(
                        x_hbm.at[idx_v.at[pl.ds((_NCH - 1) * _CH, _LASTP)]],
                        bufa.at[pl.ds(0, _LASTP)], gsem[c % 2])
                return pltpu.async_copy(
                    x_hbm.at[idx_v.at[pl.ds(c * _CH, _CH)]],
                    bufs[c % 2], gsem[c % 2])

            def scale(c):
                buf = bufs[c % 2]
                n = _LASTP if c == _NCH - 1 else _CH
                def rb(i, _):
                    for k2 in range(_D // 16):
                        sl = pl.ds(k2 * 16, 16)
                        buf[i, sl] = buf[i, sl] * _SCALE
                    return 0
                lax.fori_loop(0, n, rb, 0)

            def start_scatter(c):
                if c == _NCH - 1:
                    return pltpu.async_copy(
                        bufa.at[pl.ds(0, _LASTP)], out_hbm.at[iols[p]],
                        ssem[c % 2])
                return pltpu.async_copy(
                    bufs[c % 2], out_hbm.at[ios[p][c]], ssem[c % 2])

            hg = [None] * _NCH
            hs = [None] * _NCH
            hg[0] = start_gather(0)
            hg[1] = start_gather(1)
            hg[0].wait()
            scale(0)
            hs[0] = start_scatter(0)
            mid_work()
            for c in range(1, _NCH):
                hg[c].wait()
                scale(c)
                hs[c - 1].wait()
                hs[c] = start_scatter(c)
                if c + 1 < _NCH:
                    hg[c + 1] = start_gather(c + 1)
            hs[_NCH - 1].wait()

        b0 = wid * rows
        fetch_scores(b0, 0).wait()
        select(b0, 0)

        def pair_body(i, carry):
            ra = b0 + 2 * i
            rb_ = ra + 1

            def mid_a():
                h = fetch_scores(rb_, 1)
                h.wait()
                select(rb_, 1)
            transfers(0, mid_a)

            def mid_b():
                @pl.when(i < (rows // 2) - 1)
                def _():
                    h = fetch_scores(rb_ + 1, 0)
                    h.wait()
                    select(rb_ + 1, 0)
            transfers(1, mid_b)
            return carry

        lax.fori_loop(0, rows // 2, pair_body, 0)

    return body


_sel_gather_cache = []


def kernel(x, cls_attn, u):
    logits = cls_attn[:, 1:]
    scores = _scores_call(logits, u).reshape(_B * _P)
    xf = jnp.transpose(x, (1, 0, 2)).reshape(_N * _B, _D)
    if not _sel_gather_cache:
        _sel_gather_cache.append(_make_sel_gather())
    outf = _sel_gather_cache[0](scores, xf)
    return jnp.transpose(outf.reshape(_NSEL, _B, _D), (1, 0, 2))
